# baseline (speedup 1.0000x reference)
import functools

import jax
import jax.numpy as jnp
from jax.experimental import pallas as pl
from jax.experimental.pallas import tpu as pltpu


def _fused_body(q_ref, mask_ref, codes_ref, sim_ref, idx_ref, w_ref, *, m, k):
    tile = jax.lax.dot_general(
        q_ref[...], codes_ref[...],
        dimension_numbers=(((1,), (1,)), ((), ())),
        preferred_element_type=jnp.float32,
    )
    sim_ref[...] = tile

    bn = tile.shape[0]
    kk = tile.shape[1]
    LANES = 128
    depth = kk // LANES
    SUB = min(bn, 64)

    lane_iota = jax.lax.broadcasted_iota(jnp.int32, (1, LANES), 1).astype(jnp.float32)

    cvs = []
    cis = []
    for r0 in range(0, bn, SUB):
        neg = jnp.full((SUB, LANES), -jnp.inf, jnp.float32)
        zero = jnp.zeros((SUB, LANES), jnp.float32)
        carry0 = (neg, neg, neg, neg, zero, zero, zero, zero)

        def body(j, carry):
            a1, a2, a3, a4, i1, i2, i3, i4 = carry
            x = sim_ref[pl.ds(r0, SUB), pl.ds(j * LANES, LANES)]
            jf = j.astype(jnp.float32) if hasattr(j, "astype") else jnp.float32(j)
            ngx = (-jnp.float32(LANES)) * jf - lane_iota
            c1 = x > a1
            c2 = x > a2
            c3 = x > a3
            c4 = x > a4
            a4n = jnp.where(c3, a3, jnp.where(c4, x, a4))
            i4n = jnp.where(c3, i3, jnp.where(c4, ngx, i4))
            a3n = jnp.where(c2, a2, jnp.where(c3, x, a3))
            i3n = jnp.where(c2, i2, jnp.where(c3, ngx, i3))
            a2n = jnp.where(c1, a1, jnp.where(c2, x, a2))
            i2n = jnp.where(c1, i1, jnp.where(c2, ngx, i2))
            a1n = jnp.where(c1, x, a1)
            i1n = jnp.where(c1, ngx, i1)
            return (a1n, a2n, a3n, a4n, i1n, i2n, i3n, i4n)

        a1, a2, a3, a4, i1, i2, i3, i4 = jax.lax.fori_loop(0, depth, body, carry0)
        cvs.append(jnp.concatenate([a1, a2, a3, a4], axis=1))
        cis.append(jnp.concatenate([i1, i2, i3, i4], axis=1))

    cv = jnp.concatenate(cvs, axis=0)
    ci = jnp.concatenate(cis, axis=0)

    vals = []
    nidxs = []
    for t in range(m):
        mx = jnp.max(cv, axis=1, keepdims=True)
        cand = jnp.where(cv == mx, ci, jnp.float32(-3e38))
        gi = jnp.max(cand, axis=1, keepdims=True)
        vals.append(mx)
        nidxs.append(gi)
        if t + 1 < m:
            cv = jnp.where(cand == gi, -jnp.inf, cv)

    v = jnp.concatenate(vals, axis=1)
    ii = (-jnp.concatenate(nidxs, axis=1)).astype(jnp.int32)

    e = jnp.exp(v - v[:, :1])
    w = e / jnp.sum(e, axis=1, keepdims=True)

    mrow = mask_ref[...]
    w = w * mrow
    iota_m = jax.lax.broadcasted_iota(jnp.int32, (bn, m), 1)
    ii = jnp.where(mrow == 0.0, iota_m, ii)

    idx_ref[...] = ii
    w_ref[...] = w


def _run(q, mask, codes, top_m):
    B, N, D = q.shape
    K = codes.shape[0]
    BN = B * N
    M = 4

    bn = 256
    while BN % bn:
        bn //= 2

    q2 = q.reshape(BN, D)
    mask2 = mask.reshape(BN, 1)

    grid = (BN // bn,)
    sim, idx, w = pl.pallas_call(
        functools.partial(_fused_body, m=M, k=K),
        grid=grid,
        in_specs=[
            pl.BlockSpec((bn, D), lambda i: (i, 0)),
            pl.BlockSpec((bn, 1), lambda i: (i, 0)),
            pl.BlockSpec((K, D), lambda i: (0, 0)),
        ],
        out_specs=[
            pl.BlockSpec((bn, K), lambda i: (i, 0)),
            pl.BlockSpec((bn, M), lambda i: (i, 0)),
            pl.BlockSpec((bn, M), lambda i: (i, 0)),
        ],
        out_shape=[
            jax.ShapeDtypeStruct((BN, K), jnp.float32),
            jax.ShapeDtypeStruct((BN, M), jnp.int32),
            jax.ShapeDtypeStruct((BN, M), jnp.float32),
        ],
        compiler_params=pltpu.CompilerParams(
            dimension_semantics=("parallel",),
        ),
    )(q2, mask2, codes)

    weights = w + (jnp.asarray(top_m) * 0).astype(w.dtype)
    return idx.reshape(B, N, M), weights.reshape(B, N, M), sim.reshape(B, N, K)


def kernel(q, mask, codes, top_m):
    return _run(q, mask, codes, top_m)


# speedup vs baseline: 1.6305x; 1.6305x over previous
import functools

import jax
import jax.numpy as jnp
from jax.experimental import pallas as pl
from jax.experimental.pallas import tpu as pltpu


def _fused_body(q_ref, mask_ref, codes_ref, sim_ref, idx_ref, w_ref, wk_ref, *, m, k):
    tile = jax.lax.dot_general(
        q_ref[...], codes_ref[...],
        dimension_numbers=(((1,), (1,)), ((), ())),
        preferred_element_type=jnp.float32,
    )
    sim_ref[...] = tile

    bn = tile.shape[0]
    niota = (-jax.lax.broadcasted_iota(jnp.int32, (1, tile.shape[1]), 1)).astype(jnp.float32)
    vals = []
    nidxs = []
    for t in range(m):
        work = tile if t == 0 else wk_ref[...]
        mx = jnp.max(work, axis=1, keepdims=True)
        cand = jnp.where(work == mx, niota, jnp.float32(-3e38))
        gi = jnp.max(cand, axis=1, keepdims=True)
        vals.append(mx)
        nidxs.append(gi)
        if t + 1 < m:
            wk_ref[...] = jnp.where(cand == gi, -jnp.inf, work)

    v = jnp.concatenate(vals, axis=1)
    ii = (-jnp.concatenate(nidxs, axis=1)).astype(jnp.int32)

    e = jnp.exp(v - v[:, :1])
    w = e / jnp.sum(e, axis=1, keepdims=True)

    mrow = mask_ref[...]
    w = w * mrow
    iota_m = jax.lax.broadcasted_iota(jnp.int32, (bn, m), 1)
    ii = jnp.where(mrow == 0.0, iota_m, ii)

    idx_ref[...] = ii
    w_ref[...] = w


def _run(q, mask, codes, top_m):
    B, N, D = q.shape
    K = codes.shape[0]
    BN = B * N
    M = 4

    bn = 256
    while BN % bn:
        bn //= 2

    q2 = q.reshape(BN, D)
    mask2 = mask.reshape(BN, 1)

    grid = (BN // bn,)
    sim, idx, w = pl.pallas_call(
        functools.partial(_fused_body, m=M, k=K),
        grid=grid,
        in_specs=[
            pl.BlockSpec((bn, D), lambda i: (i, 0)),
            pl.BlockSpec((bn, 1), lambda i: (i, 0)),
            pl.BlockSpec((K, D), lambda i: (0, 0)),
        ],
        out_specs=[
            pl.BlockSpec((bn, K), lambda i: (i, 0)),
            pl.BlockSpec((bn, M), lambda i: (i, 0)),
            pl.BlockSpec((bn, M), lambda i: (i, 0)),
        ],
        out_shape=[
            jax.ShapeDtypeStruct((BN, K), jnp.float32),
            jax.ShapeDtypeStruct((BN, M), jnp.int32),
            jax.ShapeDtypeStruct((BN, M), jnp.float32),
        ],
        scratch_shapes=[pltpu.VMEM((bn, K), jnp.float32)],
        compiler_params=pltpu.CompilerParams(
            dimension_semantics=("parallel",),
        ),
    )(q2, mask2, codes)

    weights = w + (jnp.asarray(top_m) * 0).astype(w.dtype)
    return idx.reshape(B, N, M), weights.reshape(B, N, M), sim.reshape(B, N, K)


def kernel(q, mask, codes, top_m):
    return _run(q, mask, codes, top_m)
